# baseline (speedup 1.0000x reference)
import functools

import jax
import jax.numpy as jnp
from jax import lax
from jax.experimental import pallas as pl
from jax.experimental.pallas import tpu as pltpu
from jax.experimental.pallas import tpu_sc as plsc

_N = 10000
_E = 320000
_D = 128
_NC = 2
_NS = 16
_NW = _NC * _NS
_CHUNK = 128
_NCHUNK = 79
_EPT = _NCHUNK * _CHUNK
_E_PAD = _NW * _EPT
_N_ACC = 10112
_ROWS_PT = _N_ACC // _NS
_DEG_ACC = 10240
_DEG_PT = _DEG_ACC // _NS
_BR = 2000
_GRID = _N // _BR
_NBUF = 2
_RING = 4

_sc_mesh = plsc.VectorSubcoreMesh(core_axis_name="c", subcore_axis_name="s")


@functools.partial(
    pl.kernel,
    out_type=[
        jax.ShapeDtypeStruct((_DEG_ACC,), jnp.float32),
        jax.ShapeDtypeStruct((_DEG_ACC,), jnp.float32),
    ],
    mesh=_sc_mesh,
    scratch_types=[
        pltpu.VMEM((_NCHUNK, 2, _CHUNK), jnp.int32),
        pltpu.VMEM((_CHUNK,), jnp.float32),
        pltpu.VMEM((_DEG_PT,), jnp.float32),
        pltpu.VMEM_SHARED((_DEG_ACC,), jnp.float32),
    ],
)
def _deg_sc(idx3, deg0, deg1, idx_v, ones_v, zbuf, deg_sh):
    c = lax.axis_index("c")
    s = lax.axis_index("s")
    wid = s * _NC + c
    pltpu.sync_copy(idx3.at[wid], idx_v)
    for i in range(_CHUNK // 16):
        ones_v[pl.ds(i * 16, 16)] = jnp.ones((16,), jnp.float32)
    for i in range(_DEG_PT // 16):
        zbuf[pl.ds(i * 16, 16)] = jnp.zeros((16,), jnp.float32)
    pltpu.sync_copy(zbuf, deg_sh.at[pl.ds(s * _DEG_PT, _DEG_PT)])
    plsc.subcore_barrier()

    def body(g, carry):
        pltpu.sync_copy(ones_v, deg_sh.at[idx_v.at[g, 1]], add=True)
        return carry

    lax.fori_loop(0, _NCHUNK, body, 0)
    plsc.subcore_barrier()
    sl = pl.ds(s * _DEG_PT, _DEG_PT)

    @pl.when(c == 0)
    def _():
        pltpu.sync_copy(deg_sh.at[sl], deg0.at[sl])

    @pl.when(c == 1)
    def _():
        pltpu.sync_copy(deg_sh.at[sl], deg1.at[sl])


@functools.partial(
    pl.kernel,
    out_type=[
        jax.ShapeDtypeStruct((_N_ACC, _D), jnp.float32),
        jax.ShapeDtypeStruct((_N_ACC, _D), jnp.float32),
    ],
    mesh=_sc_mesh,
    scratch_types=[
        pltpu.VMEM((_RING, 2, _CHUNK), jnp.int32),
        [pltpu.VMEM((_CHUNK, _D), jnp.float32) for _ in range(_NBUF)],
        pltpu.VMEM_SHARED((_N_ACC, _D), jnp.float32),
        [pltpu.SemaphoreType.DMA for _ in range(2 * _NBUF)],
        [pltpu.SemaphoreType.DMA for _ in range(_RING)],
    ],
)
def _agg_sc(hs, idx3, zrows, out0, out1, iring, bufs, acc_sh, gsems, isems):
    c = lax.axis_index("c")
    s = lax.axis_index("s")
    wid = s * _NC + c
    pltpu.sync_copy(zrows, acc_sh.at[pl.ds(s * _ROWS_PT, _ROWS_PT)])
    plsc.subcore_barrier()

    _HC = _CHUNK // 2

    def _fire_gather(slot, b):
        pltpu.async_copy(hs.at[iring.at[slot, 0, pl.ds(0, _HC)]],
                         bufs[b].at[pl.ds(0, _HC)], gsems[2 * b])
        pltpu.async_copy(hs.at[iring.at[slot, 0, pl.ds(_HC, _HC)]],
                         bufs[b].at[pl.ds(_HC, _HC)], gsems[2 * b + 1])

    def _wait_gather(b):
        for k in range(2):
            pltpu.make_async_copy(
                hs.at[iring.at[0, 0, pl.ds(0, _HC)]],
                bufs[b].at[pl.ds(0, _HC)], gsems[2 * b + k]).wait()

    for r in range(_RING):
        pltpu.async_copy(idx3.at[wid, r], iring.at[r], isems[r])
    for b in range(_NBUF):
        pltpu.make_async_copy(idx3.at[wid, b], iring.at[b], isems[b]).wait()
        _fire_gather(b, b)

    def body(i, carry):
        g = i * _RING
        for r in range(_RING):
            gr = g + r
            b = r % _NBUF
            r2 = (r + _NBUF) % _RING

            @pl.when(gr < _NCHUNK)
            def _():
                _wait_gather(b)
                pltpu.sync_copy(bufs[b], acc_sh.at[iring.at[r, 1]], add=True)

                @pl.when(gr + _RING < _NCHUNK)
                def _():
                    pltpu.async_copy(
                        idx3.at[wid, gr + _RING], iring.at[r], isems[r])

                @pl.when(gr + _NBUF < _NCHUNK)
                def _():
                    pltpu.make_async_copy(
                        idx3.at[wid, 0], iring.at[r2], isems[r2]).wait()
                    _fire_gather(r2, b)

        return carry

    lax.fori_loop(0, (_NCHUNK + _RING - 1) // _RING, body, 0)
    plsc.subcore_barrier()
    sl = pl.ds(s * _ROWS_PT, _ROWS_PT)

    @pl.when(c == 0)
    def _():
        pltpu.sync_copy(acc_sh.at[sl], out0.at[sl])

    @pl.when(c == 1)
    def _():
        pltpu.sync_copy(acc_sh.at[sl], out1.at[sl])


def _first_body(x_ref, w_ref, d0_ref, d1_ref, hs_ref, dinv_ref):
    deg = d0_ref[...] + d1_ref[...] + 1.0
    dinv = lax.rsqrt(deg)
    hl = jnp.dot(x_ref[...], w_ref[...], preferred_element_type=jnp.float32)
    hs_ref[...] = hl * dinv
    dinv_ref[...] = dinv


def _tc_first(x, W1, d0, d1):
    return pl.pallas_call(
        _first_body,
        grid=(_GRID,),
        in_specs=[
            pl.BlockSpec((_BR, _D), lambda i: (i, 0)),
            pl.BlockSpec((_D, _D), lambda i: (0, 0)),
            pl.BlockSpec((_BR, 1), lambda i: (i, 0)),
            pl.BlockSpec((_BR, 1), lambda i: (i, 0)),
        ],
        out_specs=[
            pl.BlockSpec((_BR, _D), lambda i: (i, 0)),
            pl.BlockSpec((_BR, 1), lambda i: (i, 0)),
        ],
        out_shape=[
            jax.ShapeDtypeStruct((_N, _D), jnp.float32),
            jax.ShapeDtypeStruct((_N, 1), jnp.float32),
        ],
    )(x, W1, d0, d1)


def _layer_body(a0_ref, a1_ref, hs_ref, dinv_ref, b_ref, w_ref, hsout_ref):
    dinv = dinv_ref[...]
    h = (a0_ref[...] + a1_ref[...] + hs_ref[...]) * dinv + b_ref[...]
    h = jnp.where(h >= 0, h, 0.01 * h)
    hl = jnp.dot(h, w_ref[...], preferred_element_type=jnp.float32)
    hsout_ref[...] = hl * dinv


def _tc_layer(a0, a1, hs, dinv, b, W):
    return pl.pallas_call(
        _layer_body,
        grid=(_GRID,),
        in_specs=[
            pl.BlockSpec((_BR, _D), lambda i: (i, 0)),
            pl.BlockSpec((_BR, _D), lambda i: (i, 0)),
            pl.BlockSpec((_BR, _D), lambda i: (i, 0)),
            pl.BlockSpec((_BR, 1), lambda i: (i, 0)),
            pl.BlockSpec((1, _D), lambda i: (0, 0)),
            pl.BlockSpec((_D, _D), lambda i: (0, 0)),
        ],
        out_specs=pl.BlockSpec((_BR, _D), lambda i: (i, 0)),
        out_shape=jax.ShapeDtypeStruct((_N, _D), jnp.float32),
    )(a0, a1, hs, dinv, b, W)


def _final_body(a0_ref, a1_ref, hs_ref, dinv_ref, b_ref, lwh_ref, lint_ref,
                uw_ref, lb_ref, out_ref, acc_ref):
    i = pl.program_id(0)
    dinv = dinv_ref[...]
    h = (a0_ref[...] + a1_ref[...] + hs_ref[...]) * dinv + b_ref[...]
    h = jnp.where(h >= 0, h, 0.01 * h)

    @pl.when(i == 0)
    def _():
        acc_ref[...] = jnp.zeros_like(acc_ref)

    acc_ref[...] += jnp.sum(h, axis=0, keepdims=True)

    @pl.when(i == _GRID - 1)
    def _():
        g = acc_ref[...] * (1.0 / _N)
        val = (jnp.sum(g * lwh_ref[...])
               + jnp.sum(uw_ref[...] * lint_ref[...]) + lb_ref[0, 0])
        out_ref[...] = jnp.full((1, 1), val, jnp.float32)


def _tc_final(a0, a1, hs, dinv, b, lwh, lint, uw, lb):
    return pl.pallas_call(
        _final_body,
        grid=(_GRID,),
        in_specs=[
            pl.BlockSpec((_BR, _D), lambda i: (i, 0)),
            pl.BlockSpec((_BR, _D), lambda i: (i, 0)),
            pl.BlockSpec((_BR, _D), lambda i: (i, 0)),
            pl.BlockSpec((_BR, 1), lambda i: (i, 0)),
            pl.BlockSpec((1, _D), lambda i: (0, 0)),
            pl.BlockSpec((1, _D), lambda i: (0, 0)),
            pl.BlockSpec((1, 2), lambda i: (0, 0)),
            pl.BlockSpec((1, 2), lambda i: (0, 0)),
            pl.BlockSpec((1, 1), lambda i: (0, 0)),
        ],
        out_specs=pl.BlockSpec((1, 1), lambda i: (0, 0)),
        out_shape=jax.ShapeDtypeStruct((1, 1), jnp.float32),
        scratch_shapes=[pltpu.VMEM((1, _D), jnp.float32)],
    )(a0, a1, hs, dinv, b, lwh, lint, uw, lb)


def kernel(x, edge_index, u, w, W1, b1, W2, b2, W3, b3, W4, b4, W5, b5,
           lin_W, lin_b):
    pad = _E_PAD - _E
    src3 = jnp.concatenate(
        [edge_index[0], jnp.zeros((pad,), jnp.int32)]).reshape(
            _NW, _NCHUNK, _CHUNK)
    dst3 = jnp.concatenate(
        [edge_index[1], jnp.full((pad,), _N, jnp.int32)]).reshape(
            _NW, _NCHUNK, _CHUNK)
    idx3 = jnp.stack([src3, dst3], axis=2)
    zrows = jnp.zeros((_ROWS_PT, _D), jnp.float32)

    deg0, deg1 = _deg_sc(idx3)
    d0 = deg0.reshape(_DEG_ACC, 1)
    d1 = deg1.reshape(_DEG_ACC, 1)

    Ws = (W1, W2, W3, W4, W5)
    bs = (b1, b2, b3, b4, b5)
    hs, dinv = _tc_first(x, W1, d0, d1)
    a0 = a1 = None
    for l in range(5):
        a0, a1 = _agg_sc(hs, idx3, zrows)
        if l < 4:
            hs = _tc_layer(a0, a1, hs, dinv,
                           bs[l].reshape(1, _D), Ws[l + 1])
    out = _tc_final(a0, a1, hs, dinv, bs[4].reshape(1, _D),
                    lin_W[:_D].reshape(1, _D), lin_W[_D:].reshape(1, 2),
                    jnp.stack([u, w]).astype(jnp.float32).reshape(1, 2),
                    lin_b.reshape(1, 1))
    return out.reshape(1)


# speedup vs baseline: 1.0046x; 1.0046x over previous
import functools

import jax
import jax.numpy as jnp
from jax import lax
from jax.experimental import pallas as pl
from jax.experimental.pallas import tpu as pltpu
from jax.experimental.pallas import tpu_sc as plsc

_N = 10000
_E = 320000
_D = 128
_NC = 2
_NS = 16
_NW = _NC * _NS
_CHUNK = 128
_NCHUNK = 79
_EPT = _NCHUNK * _CHUNK
_E_PAD = _NW * _EPT
_N_ACC = 10112
_ROWS_PT = _N_ACC // _NS
_DEG_ACC = 10240
_DEG_PT = _DEG_ACC // _NS
_BR = 2000
_GRID = _N // _BR
_NBUF = 2
_RING = 4

_sc_mesh = plsc.VectorSubcoreMesh(core_axis_name="c", subcore_axis_name="s")


@functools.partial(
    pl.kernel,
    out_type=[
        jax.ShapeDtypeStruct((_DEG_ACC,), jnp.float32),
        jax.ShapeDtypeStruct((_DEG_ACC,), jnp.float32),
    ],
    mesh=_sc_mesh,
    scratch_types=[
        pltpu.VMEM((_NCHUNK, 2, _CHUNK), jnp.int32),
        pltpu.VMEM((_CHUNK,), jnp.float32),
        pltpu.VMEM((_DEG_PT,), jnp.float32),
        pltpu.VMEM_SHARED((_DEG_ACC,), jnp.float32),
    ],
)
def _deg_sc(idx3, deg0, deg1, idx_v, ones_v, zbuf, deg_sh):
    c = lax.axis_index("c")
    s = lax.axis_index("s")
    wid = s * _NC + c
    pltpu.sync_copy(idx3.at[wid], idx_v)
    for i in range(_CHUNK // 16):
        ones_v[pl.ds(i * 16, 16)] = jnp.ones((16,), jnp.float32)
    for i in range(_DEG_PT // 16):
        zbuf[pl.ds(i * 16, 16)] = jnp.zeros((16,), jnp.float32)
    pltpu.sync_copy(zbuf, deg_sh.at[pl.ds(s * _DEG_PT, _DEG_PT)])
    plsc.subcore_barrier()

    def body(g, carry):
        pltpu.sync_copy(ones_v, deg_sh.at[idx_v.at[g, 1]], add=True)
        return carry

    lax.fori_loop(0, _NCHUNK, body, 0)
    plsc.subcore_barrier()
    sl = pl.ds(s * _DEG_PT, _DEG_PT)

    @pl.when(c == 0)
    def _():
        pltpu.sync_copy(deg_sh.at[sl], deg0.at[sl])

    @pl.when(c == 1)
    def _():
        pltpu.sync_copy(deg_sh.at[sl], deg1.at[sl])


@functools.partial(
    pl.kernel,
    out_type=[
        jax.ShapeDtypeStruct((_N_ACC, _D), jnp.float32),
        jax.ShapeDtypeStruct((_N_ACC, _D), jnp.float32),
    ],
    mesh=_sc_mesh,
    scratch_types=[
        pltpu.VMEM((_RING, 2, _CHUNK), jnp.int32),
        [pltpu.VMEM((_CHUNK, _D), jnp.float32) for _ in range(_NBUF)],
        pltpu.VMEM_SHARED((_N_ACC, _D), jnp.float32),
        [pltpu.SemaphoreType.DMA for _ in range(_NBUF)],
        [pltpu.SemaphoreType.DMA for _ in range(_RING)],
    ],
)
def _agg_sc(hs, idx3, zrows, out0, out1, iring, bufs, acc_sh, gsems, isems):
    c = lax.axis_index("c")
    s = lax.axis_index("s")
    wid = s * _NC + c
    pltpu.sync_copy(zrows, acc_sh.at[pl.ds(s * _ROWS_PT, _ROWS_PT)])
    plsc.subcore_barrier()

    for r in range(_RING):
        pltpu.async_copy(idx3.at[wid, r], iring.at[r], isems[r])
    for b in range(_NBUF):
        pltpu.make_async_copy(idx3.at[wid, b], iring.at[b], isems[b]).wait()
        pltpu.async_copy(hs.at[iring.at[b, 0]], bufs[b], gsems[b])

    def body(i, carry):
        g = i * _RING
        for r in range(_RING):
            gr = g + r
            b = r % _NBUF
            r2 = (r + _NBUF) % _RING

            @pl.when(gr < _NCHUNK)
            def _():
                pltpu.make_async_copy(
                    hs.at[iring.at[r, 0]], bufs[b], gsems[b]).wait()
                pltpu.sync_copy(bufs[b], acc_sh.at[iring.at[r, 1]], add=True)

                @pl.when(gr + _RING < _NCHUNK)
                def _():
                    pltpu.async_copy(
                        idx3.at[wid, gr + _RING], iring.at[r], isems[r])

                @pl.when(gr + _NBUF < _NCHUNK)
                def _():
                    pltpu.make_async_copy(
                        idx3.at[wid, 0], iring.at[r2], isems[r2]).wait()
                    pltpu.async_copy(
                        hs.at[iring.at[r2, 0]], bufs[b], gsems[b])

        return carry

    lax.fori_loop(0, (_NCHUNK + _RING - 1) // _RING, body, 0)
    plsc.subcore_barrier()
    sl = pl.ds(s * _ROWS_PT, _ROWS_PT)

    @pl.when(c == 0)
    def _():
        pltpu.sync_copy(acc_sh.at[sl], out0.at[sl])

    @pl.when(c == 1)
    def _():
        pltpu.sync_copy(acc_sh.at[sl], out1.at[sl])


def _first_body(x_ref, w_ref, d0_ref, d1_ref, hs_ref, dinv_ref):
    deg = d0_ref[...] + d1_ref[...] + 1.0
    dinv = lax.rsqrt(deg)
    hl = jnp.dot(x_ref[...], w_ref[...], preferred_element_type=jnp.float32)
    hs_ref[...] = hl * dinv
    dinv_ref[...] = dinv


def _tc_first(x, W1, d0, d1):
    return pl.pallas_call(
        _first_body,
        grid=(_GRID,),
        in_specs=[
            pl.BlockSpec((_BR, _D), lambda i: (i, 0)),
            pl.BlockSpec((_D, _D), lambda i: (0, 0)),
            pl.BlockSpec((_BR, 1), lambda i: (i, 0)),
            pl.BlockSpec((_BR, 1), lambda i: (i, 0)),
        ],
        out_specs=[
            pl.BlockSpec((_BR, _D), lambda i: (i, 0)),
            pl.BlockSpec((_BR, 1), lambda i: (i, 0)),
        ],
        out_shape=[
            jax.ShapeDtypeStruct((_N, _D), jnp.float32),
            jax.ShapeDtypeStruct((_N, 1), jnp.float32),
        ],
    )(x, W1, d0, d1)


def _layer_body(a0_ref, a1_ref, hs_ref, dinv_ref, b_ref, w_ref, hsout_ref):
    dinv = dinv_ref[...]
    h = (a0_ref[...] + a1_ref[...] + hs_ref[...]) * dinv + b_ref[...]
    h = jnp.where(h >= 0, h, 0.01 * h)
    hl = jnp.dot(h, w_ref[...], preferred_element_type=jnp.float32)
    hsout_ref[...] = hl * dinv


def _tc_layer(a0, a1, hs, dinv, b, W):
    return pl.pallas_call(
        _layer_body,
        grid=(_GRID,),
        in_specs=[
            pl.BlockSpec((_BR, _D), lambda i: (i, 0)),
            pl.BlockSpec((_BR, _D), lambda i: (i, 0)),
            pl.BlockSpec((_BR, _D), lambda i: (i, 0)),
            pl.BlockSpec((_BR, 1), lambda i: (i, 0)),
            pl.BlockSpec((1, _D), lambda i: (0, 0)),
            pl.BlockSpec((_D, _D), lambda i: (0, 0)),
        ],
        out_specs=pl.BlockSpec((_BR, _D), lambda i: (i, 0)),
        out_shape=jax.ShapeDtypeStruct((_N, _D), jnp.float32),
    )(a0, a1, hs, dinv, b, W)


def _final_body(a0_ref, a1_ref, hs_ref, dinv_ref, b_ref, lwh_ref, lint_ref,
                uw_ref, lb_ref, out_ref, acc_ref):
    i = pl.program_id(0)
    dinv = dinv_ref[...]
    h = (a0_ref[...] + a1_ref[...] + hs_ref[...]) * dinv + b_ref[...]
    h = jnp.where(h >= 0, h, 0.01 * h)

    @pl.when(i == 0)
    def _():
        acc_ref[...] = jnp.zeros_like(acc_ref)

    acc_ref[...] += jnp.sum(h, axis=0, keepdims=True)

    @pl.when(i == _GRID - 1)
    def _():
        g = acc_ref[...] * (1.0 / _N)
        val = (jnp.sum(g * lwh_ref[...])
               + jnp.sum(uw_ref[...] * lint_ref[...]) + lb_ref[0, 0])
        out_ref[...] = jnp.full((1, 1), val, jnp.float32)


def _tc_final(a0, a1, hs, dinv, b, lwh, lint, uw, lb):
    return pl.pallas_call(
        _final_body,
        grid=(_GRID,),
        in_specs=[
            pl.BlockSpec((_BR, _D), lambda i: (i, 0)),
            pl.BlockSpec((_BR, _D), lambda i: (i, 0)),
            pl.BlockSpec((_BR, _D), lambda i: (i, 0)),
            pl.BlockSpec((_BR, 1), lambda i: (i, 0)),
            pl.BlockSpec((1, _D), lambda i: (0, 0)),
            pl.BlockSpec((1, _D), lambda i: (0, 0)),
            pl.BlockSpec((1, 2), lambda i: (0, 0)),
            pl.BlockSpec((1, 2), lambda i: (0, 0)),
            pl.BlockSpec((1, 1), lambda i: (0, 0)),
        ],
        out_specs=pl.BlockSpec((1, 1), lambda i: (0, 0)),
        out_shape=jax.ShapeDtypeStruct((1, 1), jnp.float32),
        scratch_shapes=[pltpu.VMEM((1, _D), jnp.float32)],
    )(a0, a1, hs, dinv, b, lwh, lint, uw, lb)


def kernel(x, edge_index, u, w, W1, b1, W2, b2, W3, b3, W4, b4, W5, b5,
           lin_W, lin_b):
    pad = _E_PAD - _E
    src3 = jnp.concatenate(
        [edge_index[0], jnp.zeros((pad,), jnp.int32)]).reshape(
            _NW, _NCHUNK, _CHUNK)
    dst3 = jnp.concatenate(
        [edge_index[1], jnp.full((pad,), _N, jnp.int32)]).reshape(
            _NW, _NCHUNK, _CHUNK)
    idx3 = jnp.stack([src3, dst3], axis=2)
    zrows = jnp.zeros((_ROWS_PT, _D), jnp.float32)

    deg0, deg1 = _deg_sc(idx3)
    d0 = deg0.reshape(_DEG_ACC, 1)
    d1 = deg1.reshape(_DEG_ACC, 1)

    Ws = (W1, W2, W3, W4, W5)
    bs = (b1, b2, b3, b4, b5)
    hs, dinv = _tc_first(x, W1, d0, d1)
    a0 = a1 = None
    for l in range(5):
        a0, a1 = _agg_sc(hs, idx3, zrows)
        if l < 4:
            hs = _tc_layer(a0, a1, hs, dinv,
                           bs[l].reshape(1, _D), Ws[l + 1])
    out = _tc_final(a0, a1, hs, dinv, bs[4].reshape(1, _D),
                    lin_W[:_D].reshape(1, _D), lin_W[_D:].reshape(1, 2),
                    jnp.stack([u, w]).astype(jnp.float32).reshape(1, 2),
                    lin_b.reshape(1, 1))
    return out.reshape(1)
